# baseline (speedup 1.0000x reference)
import functools

import jax
import jax.numpy as jnp
from jax import lax
from jax.experimental import pallas as pl
from jax.experimental.pallas import tpu as pltpu
from jax.experimental.pallas import tpu_sc as plsc

_S15 = 15.0 ** 0.5
_S5 = 5.0 ** 0.5
_SH_NORM = 1.0 / (4.0 * jnp.pi) ** 0.5

_K = 64
_CW = 128
_R = 10368
_TRASH = 10240
_EK = 2048


def _scales_body(v_ref, out_ref, *, n_edges, ek):
    i = pl.program_id(0)
    v = v_ref[...]
    vx = v[0:1, :]
    vy = v[1:2, :]
    vz = v[2:3, :]
    norm = jnp.sqrt(vx * vx + vy * vy + vz * vz)
    inv = 1.0 / jnp.maximum(norm, 1e-12)
    nx = vx * inv
    ny = vy * inv
    nz = vz * inv
    sh0 = (_S15 * _SH_NORM) * nx * nz
    sh1 = (_S15 * _SH_NORM) * nx * ny
    sh2 = _S5 * _SH_NORM * (ny * ny - 0.5 * (nx * nx + nz * nz))
    sh3 = (_S15 * _SH_NORM) * ny * nz
    sh4 = (0.5 * _S15 * _SH_NORM) * (nz * nz - nx * nx)
    one = jnp.ones_like(vx)
    cols = i * ek + jax.lax.broadcasted_iota(jnp.int32, (1, ek), 1)
    valid = (cols < n_edges).astype(jnp.float32)
    out_ref[...] = jnp.concatenate(
        [one, sh0, sh1, sh2, sh3, sh4, one, one], axis=0) * valid


def _sc_body(x_hbm, sc_hbm, ei_hbm, z_hbm, ones_hbm, out_hbm,
             x_buf, s_buf, i_buf, p_buf, table, sem_x, sem_s, sem_i, sem_o,
             *, n_edges, e_pad):
    c = lax.axis_index("c")
    s = lax.axis_index("s")
    per_sub = e_pad // 16
    rows_per = _R // 16

    for p in range(4):
        if p < 3:
            ch = p * 2 + c
            srow = ch
            base = s * per_sub
            n_chunks = per_sub // _K
        else:
            ch = 6 + c
            srow = 6
            base = (s * 2 + c) * (e_pad // 32)
            n_chunks = (e_pad // 32) // _K

        def fire_in(g, b, p=p, base=base, srow=srow):
            e0 = base + g * _K
            if p < 3:
                e0x = jnp.minimum(e0, n_edges - _K)
                pltpu.async_copy(x_hbm.at[pl.ds(e0x, _K)], x_buf.at[b], sem_x.at[b])
            else:
                pltpu.async_copy(ones_hbm.at[pl.ds(0, _K)], x_buf.at[b], sem_x.at[b])
            pltpu.async_copy(sc_hbm.at[pl.ds(srow * e_pad + e0, _K)], s_buf.at[b], sem_s.at[b])
            pltpu.async_copy(ei_hbm.at[pl.ds(e0, _K)], i_buf.at[b], sem_i.at[b])

        def wait_in(b):
            pltpu.make_async_copy(x_hbm.at[pl.ds(0, _K)], x_buf.at[b], sem_x.at[b]).wait()
            pltpu.make_async_copy(sc_hbm.at[pl.ds(0, _K)], s_buf.at[b], sem_s.at[b]).wait()
            pltpu.make_async_copy(ei_hbm.at[pl.ds(0, _K)], i_buf.at[b], sem_i.at[b]).wait()

        def compute(b):
            @pl.loop(0, _K, step=16)
            def _(k0):
                sv16 = s_buf[b, pl.ds(k0, 16)]
                for i in range(16):
                    sv = sv16[i]
                    for j in range(8):
                        sl = pl.ds(16 * j, 16)
                        p_buf[b, k0 + i, sl] = x_buf[b, k0 + i, sl] * sv

        def scatter_async(b):
            pltpu.async_copy(p_buf.at[b], table.at[i_buf.at[b]], sem_o.at[b],
                             add=True)

        def wait_scatter(b):
            pltpu.make_async_copy(p_buf.at[b], table.at[i_buf.at[b]],
                                  sem_o.at[b]).wait()

        pltpu.sync_copy(z_hbm, table.at[pl.ds(s * rows_per, rows_per)])
        plsc.subcore_barrier()

        fire_in(0, 0)
        fire_in(1, 1)
        last = n_chunks - 1

        @pl.loop(0, n_chunks, step=2)
        def _chunks(g):
            for b in range(2):
                wait_in(b)

                @pl.when(g >= 2)
                def _(b=b):
                    wait_scatter(b)

                compute(b)
                scatter_async(b)
                fire_in(jnp.minimum(g + 2 + b, last), b)

        wait_in(0)
        wait_in(1)
        wait_scatter(0)
        wait_scatter(1)

        plsc.subcore_barrier()
        pltpu.sync_copy(table.at[pl.ds(s * rows_per, rows_per)],
                        out_hbm.at[ch, pl.ds(s * rows_per, rows_per)])
        if p < 3:
            plsc.subcore_barrier()


def _silu(v):
    return v * (1.0 / (1.0 + jnp.exp(-v)))


def _mlp_body(sums_ref, batch_ref, ws0_ref, bs0_ref, ws1_ref, bs1_ref,
              wi0_ref, bi0_ref, wi1_ref, bi1_ref, out_ref, *, nb, n_nodes, n_graphs):
    j = pl.program_id(0)
    n_blocks = pl.num_programs(0)
    cnt = sums_ref[6][:, 0:1] + sums_ref[7][:, 0:1]
    inv = 1.0 / jnp.maximum(cnt, 1.0)

    ws0 = ws0_ref[...].astype(jnp.bfloat16)
    wi0 = wi0_ref[...].astype(jnp.bfloat16)
    ws1 = ws1_ref[...]
    wi1 = wi1_ref[...]
    bs0 = bs0_ref[...]
    bi0 = bi0_ref[...]
    bs1 = bs1_ref[...]
    bi1 = bi1_ref[...]

    cols = []
    ns = (sums_ref[0][:, 0:128] * inv).astype(jnp.bfloat16)
    h = _silu(jnp.dot(ns, ws0, preferred_element_type=jnp.float32) + bs0)
    cols.append(jnp.sum(h * ws1, axis=1, keepdims=True) + bs1)
    for k in range(5):
        nik = (sums_ref[k + 1][:, 0:128] * inv).astype(jnp.bfloat16)
        hk = _silu(jnp.dot(nik, wi0, preferred_element_type=jnp.float32) + bi0)
        cols.append(jnp.sum(hk * wi1, axis=1, keepdims=True) + bi1)

    rows = j * nb + jax.lax.broadcasted_iota(jnp.int32, (nb, 1), 0)
    valid = (rows < n_nodes).astype(jnp.float32)
    cols.append(valid)
    cols.append(jnp.zeros_like(valid))
    vals = jnp.concatenate(cols, axis=1) * valid

    batch = batch_ref[0]
    iota_g = jax.lax.broadcasted_iota(jnp.int32, (n_graphs, nb), 0)
    ohg = (iota_g == batch).astype(jnp.float32)
    gsum = jnp.dot(ohg, vals, preferred_element_type=jnp.float32)

    @pl.when(j == 0)
    def _init():
        out_ref[...] = gsum

    @pl.when(j != 0)
    def _acc():
        out_ref[...] += gsum

    @pl.when(j == n_blocks - 1)
    def _final():
        g = out_ref[...]
        ginv = 1.0 / jnp.maximum(g[:, 6:7], 1.0)
        out_ref[...] = g * ginv


@functools.partial(jax.jit, static_argnames=("n_graphs",))
def _run(edge_distance_vec, x_edge, Ws0, bs0, Ws1, bs1, Wi0, bi0, Wi1, bi1,
         edge_index, batch, n_graphs=64):
    e = edge_index.shape[0]
    n_nodes = batch.shape[0]
    e_pad = ((e + 4095) // 4096) * 4096

    vec8 = jnp.concatenate(
        [edge_distance_vec.T,
         jnp.zeros((5, e), jnp.float32)], axis=0)
    vec8 = jnp.concatenate([vec8, jnp.zeros((8, e_pad - e), jnp.float32)], axis=1)
    idx_pad = jnp.concatenate(
        [edge_index, jnp.full((e_pad - e,), _TRASH, jnp.int32)])

    scales = pl.pallas_call(
        functools.partial(_scales_body, n_edges=e, ek=_EK),
        grid=(e_pad // _EK,),
        in_specs=[pl.BlockSpec((8, _EK), lambda i: (0, i))],
        out_specs=pl.BlockSpec((8, _EK), lambda i: (0, i)),
        out_shape=jax.ShapeDtypeStruct((8, e_pad), jnp.float32),
    )(vec8)

    zeros_blk = jnp.zeros((_R // 16, _CW), jnp.float32)
    ones_blk = jnp.ones((_K, 128), jnp.float32)
    mesh = plsc.VectorSubcoreMesh(core_axis_name="c", subcore_axis_name="s")
    sums = pl.kernel(
        functools.partial(_sc_body, n_edges=e, e_pad=e_pad),
        out_type=jax.ShapeDtypeStruct((8, _R, _CW), jnp.float32),
        mesh=mesh,
        scratch_types=[
            pltpu.VMEM((2, _K, 128), jnp.float32),
            pltpu.VMEM((2, _K), jnp.float32),
            pltpu.VMEM((2, _K), jnp.int32),
            pltpu.VMEM((2, _K, 128), jnp.float32),
            pltpu.VMEM_SHARED((_R, _CW), jnp.float32),
            pltpu.SemaphoreType.DMA((2,)),
            pltpu.SemaphoreType.DMA((2,)),
            pltpu.SemaphoreType.DMA((2,)),
            pltpu.SemaphoreType.DMA((2,)),
        ],
    )(x_edge, scales.reshape(8 * e_pad), idx_pad, zeros_blk, ones_blk)

    nb = 1152
    n_nb = _R // nb
    batch_pad = jnp.concatenate(
        [batch, jnp.full((_R - n_nodes,), -1, jnp.int32)]).reshape(1, 1, _R)

    out = pl.pallas_call(
        functools.partial(_mlp_body, nb=nb, n_nodes=n_nodes, n_graphs=n_graphs),
        grid=(n_nb,),
        in_specs=[
            pl.BlockSpec((8, nb, _CW), lambda i: (0, i, 0)),
            pl.BlockSpec((1, 1, nb), lambda i: (0, 0, i)),
            pl.BlockSpec((128, 128), lambda i: (0, 0)),
            pl.BlockSpec((1, 128), lambda i: (0, 0)),
            pl.BlockSpec((1, 128), lambda i: (0, 0)),
            pl.BlockSpec((1, 1), lambda i: (0, 0)),
            pl.BlockSpec((128, 128), lambda i: (0, 0)),
            pl.BlockSpec((1, 128), lambda i: (0, 0)),
            pl.BlockSpec((1, 128), lambda i: (0, 0)),
            pl.BlockSpec((1, 1), lambda i: (0, 0)),
        ],
        out_specs=pl.BlockSpec((n_graphs, 8), lambda i: (0, 0)),
        out_shape=jax.ShapeDtypeStruct((n_graphs, 8), jnp.float32),
    )(sums, batch_pad, Ws0, bs0.reshape(1, 128), Ws1.reshape(1, 128),
      bs1.reshape(1, 1), Wi0, bi0.reshape(1, 128), Wi1.reshape(1, 128),
      bi1.reshape(1, 1))

    return out[:, 0], out[:, 1:6]


def kernel(edge_distance_vec, x_edge, Ws0, bs0, Ws1, bs1, Wi0, bi0, Wi1, bi1,
           edge_index, batch):
    return _run(edge_distance_vec, x_edge, Ws0, bs0, Ws1, bs1, Wi0, bi0, Wi1,
                bi1, edge_index, batch)


# speedup vs baseline: 2.5634x; 2.5634x over previous
import dataclasses
import functools

import jax
import jax.numpy as jnp
from jax import lax
from jax.experimental import pallas as pl
from jax.experimental.pallas import tpu as pltpu
from jax.experimental.pallas import tpu_sc as plsc

_S15 = 15.0 ** 0.5
_S5 = 5.0 ** 0.5
_SH_NORM = 1.0 / (4.0 * jnp.pi) ** 0.5

_K = 128
_CW = 128
_R = 10368
_TRASH = 10240
_EK = 2048


def _scales_body(v_ref, out_ref, *, n_edges, ek):
    i = pl.program_id(0)
    v = v_ref[...]
    vx = v[0:1, :]
    vy = v[1:2, :]
    vz = v[2:3, :]
    norm = jnp.sqrt(vx * vx + vy * vy + vz * vz)
    inv = 1.0 / jnp.maximum(norm, 1e-12)
    nx = vx * inv
    ny = vy * inv
    nz = vz * inv
    sh0 = (_S15 * _SH_NORM) * nx * nz
    sh1 = (_S15 * _SH_NORM) * nx * ny
    sh2 = _S5 * _SH_NORM * (ny * ny - 0.5 * (nx * nx + nz * nz))
    sh3 = (_S15 * _SH_NORM) * ny * nz
    sh4 = (0.5 * _S15 * _SH_NORM) * (nz * nz - nx * nx)
    one = jnp.ones_like(vx)
    cols = i * ek + jax.lax.broadcasted_iota(jnp.int32, (1, ek), 1)
    valid = (cols < n_edges).astype(jnp.float32)
    out_ref[...] = jnp.concatenate(
        [one, sh0, sh1, sh2, sh3, sh4, one, one], axis=0) * valid


def _sc_body(x_hbm, sc_hbm, ei_hbm, z_hbm, out_hbm, cnt_hbm,
             x_buf, s_buf, i_buf, hist, table, sem_x, sem_s, sem_i,
             *, n_edges, e_pad):
    c = lax.axis_index("c")
    s = lax.axis_index("s")
    per_sub = e_pad // 16
    rows_per = _R // 16

    zero16 = jnp.zeros((16,), jnp.float32)

    @pl.loop(0, _R, step=16)
    def _zh(k):
        hist[pl.ds(k, 16)] = zero16

    ones16 = jnp.ones((16,), jnp.float32)

    for p in range(3):
        ch = p * 2 + c
        srow = ch
        base = s * per_sub
        n_chunks = per_sub // _K

        def fire_in(g, b, base=base, srow=srow):
            e0 = base + g * _K
            e0x = jnp.minimum(e0, n_edges - _K)
            pltpu.async_copy(x_hbm.at[pl.ds(e0x, _K)], x_buf.at[b], sem_x.at[b])
            pltpu.async_copy(sc_hbm.at[pl.ds(srow * e_pad + e0, _K)], s_buf.at[b], sem_s.at[b])
            pltpu.async_copy(ei_hbm.at[pl.ds(e0, _K)], i_buf.at[b], sem_i.at[b])

        def wait_in(b):
            pltpu.make_async_copy(x_hbm.at[pl.ds(0, _K)], x_buf.at[b], sem_x.at[b]).wait()
            pltpu.make_async_copy(sc_hbm.at[pl.ds(0, _K)], s_buf.at[b], sem_s.at[b]).wait()
            pltpu.make_async_copy(ei_hbm.at[pl.ds(0, _K)], i_buf.at[b], sem_i.at[b]).wait()

        def compute(b):
            @pl.loop(0, _K, step=16)
            def _(k0):
                sv16 = s_buf[b, pl.ds(k0, 16)]
                for i in range(16):
                    sv = sv16[i]
                    for j in range(8):
                        sl = pl.ds(16 * j, 16)
                        x_buf[b, k0 + i, sl] = x_buf[b, k0 + i, sl] * sv

        def scatter(b):
            pltpu.sync_copy(x_buf.at[b], table.at[i_buf.at[b]], add=True)

        pltpu.sync_copy(z_hbm, table.at[pl.ds(s * rows_per, rows_per)])
        plsc.subcore_barrier()

        fire_in(0, 0)
        fire_in(1, 1)
        last = n_chunks - 1

        @pl.loop(0, n_chunks, step=2)
        def _chunks(g):
            for b in range(2):
                wait_in(b)
                if p == 0:
                    @pl.loop(0, _K, step=16)
                    def _hu(k0, b=b):
                        iv = i_buf[b, pl.ds(k0, 16)]
                        plsc.addupdate_scatter(hist, [iv], ones16)
                compute(b)
                scatter(b)
                fire_in(jnp.minimum(g + 2 + b, last), b)

        wait_in(0)
        wait_in(1)

        if p == 0:
            wid = s * 2 + c
            pltpu.sync_copy(hist, cnt_hbm.at[pl.ds(wid * _R, _R)])

        plsc.subcore_barrier()
        pltpu.sync_copy(table.at[pl.ds(s * rows_per, rows_per)],
                        out_hbm.at[ch, pl.ds(s * rows_per, rows_per)])
        if p < 2:
            plsc.subcore_barrier()


def _silu(v):
    return v * (1.0 / (1.0 + jnp.exp(-v)))


def _mlp_body(sums_ref, cnt_ref, batch_ref, ws0_ref, bs0_ref, ws1_ref, bs1_ref,
              wi0_ref, bi0_ref, wi1_ref, bi1_ref, out_ref, *, nb, n_nodes, n_graphs):
    j = pl.program_id(0)
    n_blocks = pl.num_programs(0)
    cnt = 0.5 * jnp.sum(cnt_ref[...], axis=1, keepdims=True)
    inv = 1.0 / jnp.maximum(cnt, 1.0)

    ws0 = ws0_ref[...].astype(jnp.bfloat16)
    wi0 = wi0_ref[...].astype(jnp.bfloat16)
    ws1 = ws1_ref[...]
    wi1 = wi1_ref[...]
    bs0 = bs0_ref[...]
    bi0 = bi0_ref[...]
    bs1 = bs1_ref[...]
    bi1 = bi1_ref[...]

    cols = []
    ns = (sums_ref[0][:, 0:128] * inv).astype(jnp.bfloat16)
    h = _silu(jnp.dot(ns, ws0, preferred_element_type=jnp.float32) + bs0)
    cols.append(jnp.sum(h * ws1, axis=1, keepdims=True) + bs1)
    for k in range(5):
        nik = (sums_ref[k + 1][:, 0:128] * inv).astype(jnp.bfloat16)
        hk = _silu(jnp.dot(nik, wi0, preferred_element_type=jnp.float32) + bi0)
        cols.append(jnp.sum(hk * wi1, axis=1, keepdims=True) + bi1)

    rows = j * nb + jax.lax.broadcasted_iota(jnp.int32, (nb, 1), 0)
    valid = (rows < n_nodes).astype(jnp.float32)
    cols.append(valid)
    cols.append(jnp.zeros_like(valid))
    vals = jnp.concatenate(cols, axis=1) * valid

    batch = batch_ref[0]
    iota_g = jax.lax.broadcasted_iota(jnp.int32, (n_graphs, nb), 0)
    ohg = (iota_g == batch).astype(jnp.float32)
    gsum = jnp.dot(ohg, vals, preferred_element_type=jnp.float32)

    @pl.when(j == 0)
    def _init():
        out_ref[...] = gsum

    @pl.when(j != 0)
    def _acc():
        out_ref[...] += gsum

    @pl.when(j == n_blocks - 1)
    def _final():
        g = out_ref[...]
        ginv = 1.0 / jnp.maximum(g[:, 6:7], 1.0)
        out_ref[...] = g * ginv


@functools.partial(jax.jit, static_argnames=("n_graphs",))
def _run(edge_distance_vec, x_edge, Ws0, bs0, Ws1, bs1, Wi0, bi0, Wi1, bi1,
         edge_index, batch, n_graphs=64):
    e = edge_index.shape[0]
    n_nodes = batch.shape[0]
    e_pad = ((e + 4095) // 4096) * 4096

    vec8 = jnp.concatenate(
        [edge_distance_vec.T,
         jnp.zeros((5, e), jnp.float32)], axis=0)
    vec8 = jnp.concatenate([vec8, jnp.zeros((8, e_pad - e), jnp.float32)], axis=1)
    idx_pad = jnp.concatenate(
        [edge_index, jnp.full((e_pad - e,), _TRASH, jnp.int32)])

    scales = pl.pallas_call(
        functools.partial(_scales_body, n_edges=e, ek=_EK),
        grid=(e_pad // _EK,),
        in_specs=[pl.BlockSpec((8, _EK), lambda i: (0, i))],
        out_specs=pl.BlockSpec((8, _EK), lambda i: (0, i)),
        out_shape=jax.ShapeDtypeStruct((8, e_pad), jnp.float32),
    )(vec8)

    zeros_blk = jnp.zeros((_R // 16, _CW), jnp.float32)
    mesh = plsc.VectorSubcoreMesh(core_axis_name="c", subcore_axis_name="s")
    cp = pltpu.CompilerParams()
    if "needs_layout_passes" in pltpu.CompilerParams.__dataclass_fields__:
        cp = dataclasses.replace(cp, needs_layout_passes=False)
    sums, cnts = pl.kernel(
        functools.partial(_sc_body, n_edges=e, e_pad=e_pad),
        out_type=[jax.ShapeDtypeStruct((6, _R, _CW), jnp.float32),
                  jax.ShapeDtypeStruct((32 * _R,), jnp.float32)],
        mesh=mesh,
        compiler_params=cp,
        scratch_types=[
            pltpu.VMEM((2, _K, 128), jnp.float32),
            pltpu.VMEM((2, _K), jnp.float32),
            pltpu.VMEM((2, _K), jnp.int32),
            pltpu.VMEM((_R,), jnp.float32),
            pltpu.VMEM_SHARED((_R, _CW), jnp.float32),
            pltpu.SemaphoreType.DMA((2,)),
            pltpu.SemaphoreType.DMA((2,)),
            pltpu.SemaphoreType.DMA((2,)),
        ],
    )(x_edge, scales.reshape(8 * e_pad), idx_pad, zeros_blk)
    cnt_t = cnts.reshape(32, _R).T

    nb = 1152
    n_nb = _R // nb
    batch_pad = jnp.concatenate(
        [batch, jnp.full((_R - n_nodes,), -1, jnp.int32)]).reshape(1, 1, _R)

    out = pl.pallas_call(
        functools.partial(_mlp_body, nb=nb, n_nodes=n_nodes, n_graphs=n_graphs),
        grid=(n_nb,),
        in_specs=[
            pl.BlockSpec((6, nb, _CW), lambda i: (0, i, 0)),
            pl.BlockSpec((nb, 32), lambda i: (i, 0)),
            pl.BlockSpec((1, 1, nb), lambda i: (0, 0, i)),
            pl.BlockSpec((128, 128), lambda i: (0, 0)),
            pl.BlockSpec((1, 128), lambda i: (0, 0)),
            pl.BlockSpec((1, 128), lambda i: (0, 0)),
            pl.BlockSpec((1, 1), lambda i: (0, 0)),
            pl.BlockSpec((128, 128), lambda i: (0, 0)),
            pl.BlockSpec((1, 128), lambda i: (0, 0)),
            pl.BlockSpec((1, 128), lambda i: (0, 0)),
            pl.BlockSpec((1, 1), lambda i: (0, 0)),
        ],
        out_specs=pl.BlockSpec((n_graphs, 8), lambda i: (0, 0)),
        out_shape=jax.ShapeDtypeStruct((n_graphs, 8), jnp.float32),
    )(sums, cnt_t, batch_pad, Ws0, bs0.reshape(1, 128), Ws1.reshape(1, 128),
      bs1.reshape(1, 1), Wi0, bi0.reshape(1, 128), Wi1.reshape(1, 128),
      bi1.reshape(1, 1))

    return out[:, 0], out[:, 1:6]


def kernel(edge_distance_vec, x_edge, Ws0, bs0, Ws1, bs1, Wi0, bi0, Wi1, bi1,
           edge_index, batch):
    return _run(edge_distance_vec, x_edge, Ws0, bs0, Ws1, bs1, Wi0, bi0, Wi1,
                bi1, edge_index, batch)
